# Initial kernel scaffold; baseline (speedup 1.0000x reference)
import jax
import jax.numpy as jnp
from jax.experimental import pallas as pl


def kernel(x, edge_index, W_l0, W_r0, b0, W_l1, W_r1, b1, W_l2, W_r2, b2):
    raise NotImplementedError("write your pallas kernel here")


# speedup vs baseline: 2.5218x; 2.5218x over previous
import functools

import jax
import jax.numpy as jnp
from jax import lax
from jax.experimental import pallas as pl
from jax.experimental.pallas import tpu as pltpu
from jax.experimental.pallas import tpu_sc as plsc

N = 10000
E = 160000
D = 256
DP = 272
NC = 2
NS = 16
HALF = N // NC
RPT = 320
HALF_PAD = NS * RPT
EPT = E // NS
CH = 80
NCHUNK = EPT // CH
BN = 1000
BR = 1000


def _pre_body(h_ref, wl_ref, wr_ref, b_ref, p_ref, q_ref):
    h = h_ref[...]
    p = jnp.dot(h, wl_ref[...], preferred_element_type=jnp.float32)
    q = jnp.dot(h, wr_ref[...], preferred_element_type=jnp.float32) + b_ref[...]
    p_ref[:, :D] = p
    col = lax.broadcasted_iota(jnp.int32, (BN, DP - D), 1)
    p_ref[:, D:] = jnp.where(col == 0, 1.0, 0.0)
    q_ref[...] = q


def _pre(h, wl, wr, b):
    return pl.pallas_call(
        _pre_body,
        grid=(N // BN,),
        in_specs=[
            pl.BlockSpec((BN, D), lambda i: (i, 0)),
            pl.BlockSpec((D, D), lambda i: (0, 0)),
            pl.BlockSpec((D, D), lambda i: (0, 0)),
            pl.BlockSpec((1, D), lambda i: (0, 0)),
        ],
        out_specs=[
            pl.BlockSpec((BN, DP), lambda i: (i, 0)),
            pl.BlockSpec((BN, D), lambda i: (i, 0)),
        ],
        out_shape=[
            jax.ShapeDtypeStruct((N, DP), jnp.float32),
            jax.ShapeDtypeStruct((N, D), jnp.float32),
        ],
    )(h, wl, wr, b.reshape(1, D))


def _comb_body(acc_ref, q_ref, o_ref, *, relu):
    a = acc_ref[0, :, :D]
    cnt = acc_ref[0, :, D:][:, :1]
    inv = 1.0 / jnp.maximum(cnt, 1.0)
    o = a * inv + q_ref[...]
    if relu:
        o = jnp.maximum(o, 0.0)
    o_ref[...] = o


def _comb(acc, q, relu):
    return pl.pallas_call(
        functools.partial(_comb_body, relu=relu),
        grid=(NC, HALF // BR),
        in_specs=[
            pl.BlockSpec((1, BR, DP), lambda c, i: (c, i, 0)),
            pl.BlockSpec((BR, D), lambda c, i: (c * (HALF // BR) + i, 0)),
        ],
        out_specs=pl.BlockSpec((BR, D), lambda c, i: (c * (HALF // BR) + i, 0)),
        out_shape=jax.ShapeDtypeStruct((N, D), jnp.float32),
    )(acc, q)


def _agg_body(p_hbm, src_hbm, dst_hbm, out_hbm, srcidx, dstidx, dstloc, rows, acc):
    c = lax.axis_index("c")
    s = lax.axis_index("s")
    zero16 = jnp.zeros((16,), jnp.float32)

    def zrow(r, carry):
        for j in range(DP // 16):
            rows[r, pl.ds(j * 16, 16)] = zero16
        return carry

    lax.fori_loop(0, CH, zrow, 0)
    base_r = s * RPT
    for z in range(RPT // CH):
        pltpu.sync_copy(rows, acc.at[pl.ds(base_r + z * CH, CH)])
    plsc.subcore_barrier()

    ebase = s * EPT
    half_base = c * HALF

    def chunk(i, carry):
        off = ebase + i * CH
        pltpu.sync_copy(src_hbm.at[pl.ds(off, CH)], srcidx)
        pltpu.sync_copy(dst_hbm.at[pl.ds(off, CH)], dstidx)
        pltpu.sync_copy(p_hbm.at[srcidx], rows)
        for g in range(CH // 16):
            dv = dstidx[pl.ds(g * 16, 16)]
            loc = dv - half_base
            ok = (loc >= 0) & (loc < HALF)
            dstloc[pl.ds(g * 16, 16)] = jnp.where(ok, loc, HALF)
        pltpu.sync_copy(rows, acc.at[dstloc], add=True)
        return carry

    lax.fori_loop(0, NCHUNK, chunk, 0)
    plsc.subcore_barrier()
    pltpu.sync_copy(acc.at[pl.ds(base_r, RPT)],
                    out_hbm.at[pl.ds(c * HALF_PAD + base_r, RPT)])


_agg = pl.kernel(
    _agg_body,
    out_type=jax.ShapeDtypeStruct((NC * HALF_PAD, DP), jnp.float32),
    mesh=plsc.VectorSubcoreMesh(core_axis_name="c", subcore_axis_name="s"),
    compiler_params=pltpu.CompilerParams(use_tc_tiling_on_sc=False),
    scratch_types=[
        pltpu.VMEM((CH,), jnp.int32),
        pltpu.VMEM((CH,), jnp.int32),
        pltpu.VMEM((CH,), jnp.int32),
        pltpu.VMEM((CH, DP), jnp.float32),
        pltpu.VMEM_SHARED((HALF_PAD, DP), jnp.float32),
    ],
)


def kernel(x, edge_index, W_l0, W_r0, b0, W_l1, W_r1, b1, W_l2, W_r2, b2):
    ei = edge_index.astype(jnp.int32)
    src, dst = ei[0], ei[1]
    h = x
    for wl, wr, b, relu in ((W_l0, W_r0, b0, True),
                            (W_l1, W_r1, b1, True),
                            (W_l2, W_r2, b2, False)):
        p, q = _pre(h, wl, wr, b)
        acc = _agg(p, src, dst).reshape(NC, HALF_PAD, DP)
        h = _comb(acc, q, relu)
    return h


# speedup vs baseline: 3.0242x; 1.1992x over previous
import functools

import jax
import jax.numpy as jnp
from jax import lax
from jax.experimental import pallas as pl
from jax.experimental.pallas import tpu as pltpu
from jax.experimental.pallas import tpu_sc as plsc

N = 10000
E = 160000
D = 256
NC = 2
NS = 16
HALF = N // NC
RPT = 320
HALF_PAD = NS * RPT
EPT = E // NS
CH = 80
NCHUNK = EPT // CH
BN = 1000
BR = 1000
CW = 16

_SC_PARAMS = pltpu.CompilerParams(use_tc_tiling_on_sc=False)
_MESH = plsc.VectorSubcoreMesh(core_axis_name="c", subcore_axis_name="s")


def _pre_body(h_ref, wl_ref, wr_ref, b_ref, p_ref, q_ref):
    h = h_ref[...]
    p_ref[...] = jnp.dot(h, wl_ref[...], preferred_element_type=jnp.float32)
    q_ref[...] = (jnp.dot(h, wr_ref[...], preferred_element_type=jnp.float32)
                  + b_ref[...])


def _pre(h, wl, wr, b):
    return pl.pallas_call(
        _pre_body,
        grid=(N // BN,),
        in_specs=[
            pl.BlockSpec((BN, D), lambda i: (i, 0)),
            pl.BlockSpec((D, D), lambda i: (0, 0)),
            pl.BlockSpec((D, D), lambda i: (0, 0)),
            pl.BlockSpec((1, D), lambda i: (0, 0)),
        ],
        out_specs=[
            pl.BlockSpec((BN, D), lambda i: (i, 0)),
            pl.BlockSpec((BN, D), lambda i: (i, 0)),
        ],
        out_shape=[
            jax.ShapeDtypeStruct((N, D), jnp.float32),
            jax.ShapeDtypeStruct((N, D), jnp.float32),
        ],
    )(h, wl, wr, b.reshape(1, D))


def _comb_body(acc_ref, cnt_ref, q_ref, o_ref, *, relu):
    a = acc_ref[0]
    cnt = cnt_ref[0][:, :1]
    inv = 1.0 / jnp.maximum(cnt, 1.0)
    o = a * inv + q_ref[...]
    if relu:
        o = jnp.maximum(o, 0.0)
    o_ref[...] = o


def _comb(acc, cnt, q, relu):
    return pl.pallas_call(
        functools.partial(_comb_body, relu=relu),
        grid=(NC, HALF // BR),
        in_specs=[
            pl.BlockSpec((1, BR, D), lambda c, i: (c, i, 0)),
            pl.BlockSpec((1, BR, CW), lambda c, i: (c, i, 0)),
            pl.BlockSpec((BR, D), lambda c, i: (c * (HALF // BR) + i, 0)),
        ],
        out_specs=pl.BlockSpec((BR, D), lambda c, i: (c * (HALF // BR) + i, 0)),
        out_shape=jax.ShapeDtypeStruct((N, D), jnp.float32),
    )(acc, cnt, q)


def _dstloc_chunk(dstidx_b, dstloc_b, half_base):
    for g in range(CH // 16):
        dv = dstidx_b[pl.ds(g * 16, 16)]
        loc = dv - half_base
        ok = (loc >= 0) & (loc < HALF)
        dstloc_b[pl.ds(g * 16, 16)] = jnp.where(ok, loc, HALF)


def _agg_body(p_hbm, src_hbm, dst_hbm, out_hbm,
              srcidx0, srcidx1, dstidx0, dstidx1, dstloc0, dstloc1,
              rows0, rows1, acc, gsem0, gsem1):
    c = lax.axis_index("c")
    s = lax.axis_index("s")
    zero16 = jnp.zeros((16,), jnp.float32)

    def zrow(r, carry):
        for j in range(D // 16):
            rows0[r, pl.ds(j * 16, 16)] = zero16
        return carry

    lax.fori_loop(0, CH, zrow, 0)
    base_r = s * RPT
    for z in range(RPT // CH):
        pltpu.sync_copy(rows0, acc.at[pl.ds(base_r + z * CH, CH)])
    plsc.subcore_barrier()

    ebase = s * EPT
    half_base = c * HALF

    bufs = ((rows0, srcidx0, dstidx0, dstloc0, gsem0),
            (rows1, srcidx1, dstidx1, dstloc1, gsem1))
    pltpu.sync_copy(src_hbm.at[pl.ds(ebase, CH)], srcidx0)
    pltpu.sync_copy(dst_hbm.at[pl.ds(ebase, CH)], dstidx0)
    pltpu.async_copy(p_hbm.at[srcidx0], rows0, gsem0)

    def super_step(j, carry):
        for b in (0, 1):
            i = 2 * j + b
            rows_b, srcidx_b, dstidx_b, dstloc_b, gsem_b = bufs[b]
            rows_o, srcidx_o, dstidx_o, dstloc_o, gsem_o = bufs[1 - b]

            @pl.when(i < NCHUNK)
            def _step():
                pltpu.make_async_copy(p_hbm.at[srcidx_b], rows_b,
                                      gsem_b).wait()
                _dstloc_chunk(dstidx_b, dstloc_b, half_base)

                @pl.when(i + 1 < NCHUNK)
                def _next():
                    off = ebase + (i + 1) * CH
                    pltpu.sync_copy(src_hbm.at[pl.ds(off, CH)], srcidx_o)
                    pltpu.sync_copy(dst_hbm.at[pl.ds(off, CH)], dstidx_o)
                    pltpu.async_copy(p_hbm.at[srcidx_o], rows_o, gsem_o)

                pltpu.sync_copy(rows_b, acc.at[dstloc_b], add=True)
        return carry

    lax.fori_loop(0, (NCHUNK + 1) // 2, super_step, 0)
    plsc.subcore_barrier()
    pltpu.sync_copy(acc.at[pl.ds(base_r, RPT)],
                    out_hbm.at[pl.ds(c * HALF_PAD + base_r, RPT)])


_agg = pl.kernel(
    _agg_body,
    out_type=jax.ShapeDtypeStruct((NC * HALF_PAD, D), jnp.float32),
    mesh=_MESH,
    compiler_params=_SC_PARAMS,
    scratch_types=[
        pltpu.VMEM((CH,), jnp.int32),
        pltpu.VMEM((CH,), jnp.int32),
        pltpu.VMEM((CH,), jnp.int32),
        pltpu.VMEM((CH,), jnp.int32),
        pltpu.VMEM((CH,), jnp.int32),
        pltpu.VMEM((CH,), jnp.int32),
        pltpu.VMEM((CH, D), jnp.float32),
        pltpu.VMEM((CH, D), jnp.float32),
        pltpu.VMEM_SHARED((HALF_PAD, D), jnp.float32),
        pltpu.SemaphoreType.DMA,
        pltpu.SemaphoreType.DMA,
    ],
)


def _cnt_body(dst_hbm, out_hbm, dstidx, dstloc, ones, acc):
    c = lax.axis_index("c")
    s = lax.axis_index("s")
    zero16 = jnp.zeros((16,), jnp.float32)

    def zrow(r, carry):
        ones[r, pl.ds(0, 16)] = zero16
        return carry

    lax.fori_loop(0, CH, zrow, 0)
    base_r = s * RPT
    for z in range(RPT // CH):
        pltpu.sync_copy(ones, acc.at[pl.ds(base_r + z * CH, CH)])

    def orow(r, carry):
        ones[r, pl.ds(0, 16)] = zero16 + 1.0
        return carry

    lax.fori_loop(0, CH, orow, 0)
    plsc.subcore_barrier()

    ebase = s * EPT
    half_base = c * HALF

    def chunk(i, carry):
        pltpu.sync_copy(dst_hbm.at[pl.ds(ebase + i * CH, CH)], dstidx)
        _dstloc_chunk(dstidx, dstloc, half_base)
        pltpu.sync_copy(ones, acc.at[dstloc], add=True)
        return carry

    lax.fori_loop(0, NCHUNK, chunk, 0)
    plsc.subcore_barrier()
    pltpu.sync_copy(acc.at[pl.ds(base_r, RPT)],
                    out_hbm.at[pl.ds(c * HALF_PAD + base_r, RPT)])


_cnt = pl.kernel(
    _cnt_body,
    out_type=jax.ShapeDtypeStruct((NC * HALF_PAD, CW), jnp.float32),
    mesh=_MESH,
    compiler_params=_SC_PARAMS,
    scratch_types=[
        pltpu.VMEM((CH,), jnp.int32),
        pltpu.VMEM((CH,), jnp.int32),
        pltpu.VMEM((CH, CW), jnp.float32),
        pltpu.VMEM_SHARED((HALF_PAD, CW), jnp.float32),
    ],
)


def kernel(x, edge_index, W_l0, W_r0, b0, W_l1, W_r1, b1, W_l2, W_r2, b2):
    ei = edge_index.astype(jnp.int32)
    src, dst = ei[0], ei[1]
    cnt = _cnt(dst).reshape(NC, HALF_PAD, CW)
    h = x
    for wl, wr, b, relu in ((W_l0, W_r0, b0, True),
                            (W_l1, W_r1, b1, True),
                            (W_l2, W_r2, b2, False)):
        p, q = _pre(h, wl, wr, b)
        acc = _agg(p, src, dst).reshape(NC, HALF_PAD, D)
        h = _comb(acc, cnt, q, relu)
    return h


# speedup vs baseline: 3.9135x; 1.2941x over previous
import functools

import jax
import jax.numpy as jnp
from jax import lax
from jax.experimental import pallas as pl
from jax.experimental.pallas import tpu as pltpu
from jax.experimental.pallas import tpu_sc as plsc

N = 10000
E = 160000
D = 256
NC = 2
NS = 16
HALF = N // NC
RPT = 320
HALF_PAD = NS * RPT
EPT = E // NS
CH = 80
NCHUNK = EPT // CH
BN = 1000
BR = 1000
CW = 16
EL = 128

_SC_PARAMS = pltpu.CompilerParams(use_tc_tiling_on_sc=False)
_MESH = plsc.VectorSubcoreMesh(core_axis_name="c", subcore_axis_name="s")


def _pre_body(h_ref, wl_ref, wr_ref, b_ref, p_ref, q_ref):
    h = h_ref[...]
    p_ref[...] = jnp.dot(h, wl_ref[...], preferred_element_type=jnp.float32)
    q_ref[...] = (jnp.dot(h, wr_ref[...], preferred_element_type=jnp.float32)
                  + b_ref[...])


def _pre(h, wl, wr, b):
    return pl.pallas_call(
        _pre_body,
        grid=(N // BN,),
        in_specs=[
            pl.BlockSpec((BN, D), lambda i: (i, 0)),
            pl.BlockSpec((D, D), lambda i: (0, 0)),
            pl.BlockSpec((D, D), lambda i: (0, 0)),
            pl.BlockSpec((1, D), lambda i: (0, 0)),
        ],
        out_specs=[
            pl.BlockSpec((BN, D), lambda i: (i, 0)),
            pl.BlockSpec((BN, D), lambda i: (i, 0)),
        ],
        out_shape=[
            jax.ShapeDtypeStruct((N, D), jnp.float32),
            jax.ShapeDtypeStruct((N, D), jnp.float32),
        ],
    )(h, wl, wr, b.reshape(1, D))


def _loc_body(d_ref, o_ref):
    c = pl.program_id(0)
    dv = d_ref[...]
    loc = dv - c * HALF
    ok = (loc >= 0) & (loc < HALF)
    o_ref[...] = jnp.where(ok, loc, HALF)[None]


def _loc(dst):
    return pl.pallas_call(
        _loc_body,
        grid=(NC,),
        in_specs=[pl.BlockSpec((E // EL, EL), lambda c: (0, 0))],
        out_specs=pl.BlockSpec((1, E // EL, EL), lambda c: (c, 0, 0)),
        out_shape=jax.ShapeDtypeStruct((NC, E // EL, EL), jnp.int32),
    )(dst.reshape(E // EL, EL)).reshape(NC * E)


def _comb_body(acc_ref, cnt_ref, q_ref, o_ref, *, relu):
    a = acc_ref[0]
    cnt = cnt_ref[0][:, :1]
    inv = 1.0 / jnp.maximum(cnt, 1.0)
    o = a * inv + q_ref[...]
    if relu:
        o = jnp.maximum(o, 0.0)
    o_ref[...] = o


def _comb(acc, cnt, q, relu):
    return pl.pallas_call(
        functools.partial(_comb_body, relu=relu),
        grid=(NC, HALF // BR),
        in_specs=[
            pl.BlockSpec((1, BR, D), lambda c, i: (c, i, 0)),
            pl.BlockSpec((1, BR, CW), lambda c, i: (c, i, 0)),
            pl.BlockSpec((BR, D), lambda c, i: (c * (HALF // BR) + i, 0)),
        ],
        out_specs=pl.BlockSpec((BR, D), lambda c, i: (c * (HALF // BR) + i, 0)),
        out_shape=jax.ShapeDtypeStruct((N, D), jnp.float32),
    )(acc, cnt, q)


def _agg_body(p_hbm, src_hbm, loc_hbm, out_hbm,
              si0, si1, si2, si3, dl0, dl1, dl2, dl3, rows0, rows1, acc,
              gs0, gs1, ss0, ss1, is0, is1, is2, is3, ds0, ds1, ds2, ds3):
    c = lax.axis_index("c")
    s = lax.axis_index("s")
    zero16 = jnp.zeros((16,), jnp.float32)

    def zrow(r, carry):
        for j in range(D // 16):
            rows0[r, pl.ds(j * 16, 16)] = zero16
        return carry

    lax.fori_loop(0, CH, zrow, 0)
    base_r = s * RPT
    for z in range(RPT // CH):
        pltpu.sync_copy(rows0, acc.at[pl.ds(base_r + z * CH, CH)])
    plsc.subcore_barrier()

    ebase = s * EPT
    lbase = c * E + s * EPT
    si = (si0, si1, si2, si3)
    dl = (dl0, dl1, dl2, dl3)
    isem = (is0, is1, is2, is3)
    dsem = (ds0, ds1, ds2, ds3)
    rows = (rows0, rows1)
    gsem = (gs0, gs1)
    ssem = (ss0, ss1)

    def start_idx(i):
        k = i % 4
        ih = pltpu.async_copy(src_hbm.at[pl.ds(ebase + i * CH, CH)],
                              si[k], isem[k])
        dh = pltpu.async_copy(loc_hbm.at[pl.ds(lbase + i * CH, CH)],
                              dl[k], dsem[k])
        return ih, dh

    ih = {}
    gh = {}
    sh = {}
    ih[0] = start_idx(0)
    if NCHUNK > 1:
        ih[1] = start_idx(1)
    ih[0][0].wait()
    ih[0][1].wait()
    gh[0] = pltpu.async_copy(p_hbm.at[si[0]], rows0, gsem[0])

    for i in range(NCHUNK):
        b = i % 2
        k = i % 4
        gh[i].wait()
        if i + 2 < NCHUNK:
            ih[i + 2] = start_idx(i + 2)
        if i >= 1:
            sh[i - 1].wait()
        if i + 1 < NCHUNK:
            ih[i + 1][0].wait()
            ih[i + 1][1].wait()
            gh[i + 1] = pltpu.async_copy(p_hbm.at[si[(i + 1) % 4]],
                                         rows[1 - b], gsem[1 - b])
        sh[i] = pltpu.async_copy(rows[b], acc.at[dl[k]], ssem[b], add=True)

    sh[NCHUNK - 1].wait()
    plsc.subcore_barrier()
    pltpu.sync_copy(acc.at[pl.ds(base_r, RPT)],
                    out_hbm.at[pl.ds(c * HALF_PAD + base_r, RPT)])


_agg = pl.kernel(
    _agg_body,
    out_type=jax.ShapeDtypeStruct((NC * HALF_PAD, D), jnp.float32),
    mesh=_MESH,
    compiler_params=_SC_PARAMS,
    scratch_types=(
        [pltpu.VMEM((CH,), jnp.int32)] * 8
        + [pltpu.VMEM((CH, D), jnp.float32)] * 2
        + [pltpu.VMEM_SHARED((HALF_PAD, D), jnp.float32)]
        + [pltpu.SemaphoreType.DMA] * 12
    ),
)


def _cnt_body(loc_hbm, out_hbm, dl0, dl1, dl2, dl3, ones, acc,
              ss0, ss1, ds0, ds1, ds2, ds3):
    c = lax.axis_index("c")
    s = lax.axis_index("s")
    zero16 = jnp.zeros((16,), jnp.float32)

    def zrow(r, carry):
        ones[r, pl.ds(0, 16)] = zero16
        return carry

    lax.fori_loop(0, CH, zrow, 0)
    base_r = s * RPT
    for z in range(RPT // CH):
        pltpu.sync_copy(ones, acc.at[pl.ds(base_r + z * CH, CH)])

    def orow(r, carry):
        ones[r, pl.ds(0, 16)] = zero16 + 1.0
        return carry

    lax.fori_loop(0, CH, orow, 0)
    plsc.subcore_barrier()

    lbase = c * E + s * EPT
    dl = (dl0, dl1, dl2, dl3)
    dsem = (ds0, ds1, ds2, ds3)
    ssem = (ss0, ss1)

    def start_idx(i):
        k = i % 4
        return pltpu.async_copy(loc_hbm.at[pl.ds(lbase + i * CH, CH)],
                                dl[k], dsem[k])

    dh = {}
    sh = {}
    dh[0] = start_idx(0)
    if NCHUNK > 1:
        dh[1] = start_idx(1)

    for i in range(NCHUNK):
        b = i % 2
        k = i % 4
        dh[i].wait()
        if i + 2 < NCHUNK:
            dh[i + 2] = start_idx(i + 2)
        if i >= 1:
            sh[i - 1].wait()
        sh[i] = pltpu.async_copy(ones, acc.at[dl[k]], ssem[b], add=True)

    sh[NCHUNK - 1].wait()
    plsc.subcore_barrier()
    pltpu.sync_copy(acc.at[pl.ds(base_r, RPT)],
                    out_hbm.at[pl.ds(c * HALF_PAD + base_r, RPT)])


_cnt = pl.kernel(
    _cnt_body,
    out_type=jax.ShapeDtypeStruct((NC * HALF_PAD, CW), jnp.float32),
    mesh=_MESH,
    compiler_params=_SC_PARAMS,
    scratch_types=(
        [pltpu.VMEM((CH,), jnp.int32)] * 4
        + [pltpu.VMEM((CH, CW), jnp.float32)]
        + [pltpu.VMEM_SHARED((HALF_PAD, CW), jnp.float32)]
        + [pltpu.SemaphoreType.DMA] * 6
    ),
)


def kernel(x, edge_index, W_l0, W_r0, b0, W_l1, W_r1, b1, W_l2, W_r2, b2):
    ei = edge_index.astype(jnp.int32)
    src, dst = ei[0], ei[1]
    loc = _loc(dst)
    cnt = _cnt(loc).reshape(NC, HALF_PAD, CW)
    h = x
    for wl, wr, b, relu in ((W_l0, W_r0, b0, True),
                            (W_l1, W_r1, b1, True),
                            (W_l2, W_r2, b2, False)):
        p, q = _pre(h, wl, wr, b)
        acc = _agg(p, src, loc).reshape(NC, HALF_PAD, D)
        h = _comb(acc, cnt, q, relu)
    return h


# speedup vs baseline: 3.9614x; 1.0122x over previous
import functools

import jax
import jax.numpy as jnp
from jax import lax
from jax.experimental import pallas as pl
from jax.experimental.pallas import tpu as pltpu
from jax.experimental.pallas import tpu_sc as plsc

N = 10000
E = 160000
D = 256
NC = 2
NS = 16
HALF = N // NC
RPT = 320
HALF_PAD = NS * RPT
EPT = E // NS
CH = 80
NCHUNK = EPT // CH
NCTOT = E // CH
BN = 1000
BR = 1000
CW = 16

_SC_PARAMS = pltpu.CompilerParams(use_tc_tiling_on_sc=False)
_MESH = plsc.VectorSubcoreMesh(core_axis_name="c", subcore_axis_name="s")


def _pre_body(h_ref, wl_ref, wr_ref, b_ref, p_ref, q_ref):
    h = h_ref[...]
    p_ref[...] = jnp.dot(h, wl_ref[...], preferred_element_type=jnp.float32)
    q_ref[...] = (jnp.dot(h, wr_ref[...], preferred_element_type=jnp.float32)
                  + b_ref[...])


def _pre(h, wl, wr, b):
    return pl.pallas_call(
        _pre_body,
        grid=(N // BN,),
        in_specs=[
            pl.BlockSpec((BN, D), lambda i: (i, 0)),
            pl.BlockSpec((D, D), lambda i: (0, 0)),
            pl.BlockSpec((D, D), lambda i: (0, 0)),
            pl.BlockSpec((1, D), lambda i: (0, 0)),
        ],
        out_specs=[
            pl.BlockSpec((BN, D), lambda i: (i, 0)),
            pl.BlockSpec((BN, D), lambda i: (i, 0)),
        ],
        out_shape=[
            jax.ShapeDtypeStruct((N, D), jnp.float32),
            jax.ShapeDtypeStruct((N, D), jnp.float32),
        ],
    )(h, wl, wr, b.reshape(1, D))


def _fuse_body(acc_ref, cnt_ref, q_ref, wl_ref, wr_ref, b_ref, p_ref, q2_ref):
    a = acc_ref[0]
    cnt = cnt_ref[0][:, :1]
    inv = 1.0 / jnp.maximum(cnt, 1.0)
    h = jnp.maximum(a * inv + q_ref[...], 0.0)
    p_ref[...] = jnp.dot(h, wl_ref[...], preferred_element_type=jnp.float32)
    q2_ref[...] = (jnp.dot(h, wr_ref[...], preferred_element_type=jnp.float32)
                   + b_ref[...])


def _fuse(acc, cnt, q, wl, wr, b):
    nb = HALF // BN
    return pl.pallas_call(
        _fuse_body,
        grid=(NC, nb),
        in_specs=[
            pl.BlockSpec((1, BN, D), lambda c, i: (c, i, 0)),
            pl.BlockSpec((1, BN, CW), lambda c, i: (c, i, 0)),
            pl.BlockSpec((BN, D), lambda c, i: (c * nb + i, 0)),
            pl.BlockSpec((D, D), lambda c, i: (0, 0)),
            pl.BlockSpec((D, D), lambda c, i: (0, 0)),
            pl.BlockSpec((1, D), lambda c, i: (0, 0)),
        ],
        out_specs=[
            pl.BlockSpec((BN, D), lambda c, i: (c * nb + i, 0)),
            pl.BlockSpec((BN, D), lambda c, i: (c * nb + i, 0)),
        ],
        out_shape=[
            jax.ShapeDtypeStruct((N, D), jnp.float32),
            jax.ShapeDtypeStruct((N, D), jnp.float32),
        ],
    )(acc, cnt, q, wl, wr, b.reshape(1, D))


def _comb_body(acc_ref, cnt_ref, q_ref, o_ref):
    a = acc_ref[0]
    cnt = cnt_ref[0][:, :1]
    inv = 1.0 / jnp.maximum(cnt, 1.0)
    o_ref[...] = a * inv + q_ref[...]


def _comb(acc, cnt, q):
    return pl.pallas_call(
        _comb_body,
        grid=(NC, HALF // BR),
        in_specs=[
            pl.BlockSpec((1, BR, D), lambda c, i: (c, i, 0)),
            pl.BlockSpec((1, BR, CW), lambda c, i: (c, i, 0)),
            pl.BlockSpec((BR, D), lambda c, i: (c * (HALF // BR) + i, 0)),
        ],
        out_specs=pl.BlockSpec((BR, D), lambda c, i: (c * (HALF // BR) + i, 0)),
        out_shape=jax.ShapeDtypeStruct((N, D), jnp.float32),
    )(acc, cnt, q)


def _cidx_body(s_ref, d_ref, o_ref):
    c = pl.program_id(0)
    dv = d_ref[...]
    loc = dv - c * HALF
    ok = (loc >= 0) & (loc < HALF)
    o_ref[0, :, 0, :] = s_ref[...]
    o_ref[0, :, 1, :] = jnp.where(ok, loc, HALF)


def _cidx(src, dst):
    out = pl.pallas_call(
        _cidx_body,
        grid=(NC,),
        in_specs=[
            pl.BlockSpec((NCTOT, CH), lambda c: (0, 0)),
            pl.BlockSpec((NCTOT, CH), lambda c: (0, 0)),
        ],
        out_specs=pl.BlockSpec((1, NCTOT, 2, CH), lambda c: (c, 0, 0, 0)),
        out_shape=jax.ShapeDtypeStruct((NC, NCTOT, 2, CH), jnp.int32),
    )(src.reshape(NCTOT, CH), dst.reshape(NCTOT, CH))
    return out.reshape(NC * NCTOT, 2, CH)


def _agg_body(p_hbm, cidx_hbm, out_hbm,
              cb0, cb1, cb2, cb3, rows0, rows1, acc,
              gs0, gs1, ss0, ss1, is0, is1, is2, is3):
    c = lax.axis_index("c")
    s = lax.axis_index("s")
    zero16 = jnp.zeros((16,), jnp.float32)

    def zrow(r, carry):
        for j in range(D // 16):
            rows0[r, pl.ds(j * 16, 16)] = zero16
        return carry

    lax.fori_loop(0, CH, zrow, 0)
    base_r = s * RPT
    for z in range(RPT // CH):
        pltpu.sync_copy(rows0, acc.at[pl.ds(base_r + z * CH, CH)])
    plsc.subcore_barrier()

    cbase = c * NCTOT + s * NCHUNK
    cb = (cb0, cb1, cb2, cb3)
    isem = (is0, is1, is2, is3)
    rows = (rows0, rows1)
    gsem = (gs0, gs1)
    ssem = (ss0, ss1)

    ihd = [pltpu.make_async_copy(cidx_hbm.at[cbase + i], cb[i % 4],
                                 isem[i % 4])
           for i in range(NCHUNK)]
    ghd = [pltpu.make_async_copy(p_hbm.at[cb[i % 4].at[0]], rows[i % 2],
                                 gsem[i % 2])
           for i in range(NCHUNK)]
    shd = [pltpu.make_async_copy(rows[i % 2], acc.at[cb[i % 4].at[1]],
                                 ssem[i % 2])
           for i in range(NCHUNK)]

    ihd[0].start()
    ihd[1].start()
    ihd[0].wait()
    ghd[0].start()

    for i in range(NCHUNK):
        ghd[i].wait()
        if i + 2 < NCHUNK:
            ihd[i + 2].start()
        if i >= 1:
            shd[i - 1].wait()
        if i + 1 < NCHUNK:
            ihd[i + 1].wait()
            ghd[i + 1].start()
        shd[i].start(add=True)

    shd[NCHUNK - 1].wait()
    plsc.subcore_barrier()
    pltpu.sync_copy(acc.at[pl.ds(base_r, RPT)],
                    out_hbm.at[pl.ds(c * HALF_PAD + base_r, RPT)])


_agg = pl.kernel(
    _agg_body,
    out_type=jax.ShapeDtypeStruct((NC * HALF_PAD, D), jnp.float32),
    mesh=_MESH,
    compiler_params=_SC_PARAMS,
    scratch_types=(
        [pltpu.VMEM((2, CH), jnp.int32)] * 4
        + [pltpu.VMEM((CH, D), jnp.float32)] * 2
        + [pltpu.VMEM_SHARED((HALF_PAD, D), jnp.float32)]
        + [pltpu.SemaphoreType.DMA] * 8
    ),
)


def _cnt_body(cidx_hbm, out_hbm, cb0, cb1, cb2, cb3, ones, acc,
              ss0, ss1, is0, is1, is2, is3):
    c = lax.axis_index("c")
    s = lax.axis_index("s")
    zero16 = jnp.zeros((16,), jnp.float32)

    def zrow(r, carry):
        ones[r, pl.ds(0, 16)] = zero16
        return carry

    lax.fori_loop(0, CH, zrow, 0)
    base_r = s * RPT
    for z in range(RPT // CH):
        pltpu.sync_copy(ones, acc.at[pl.ds(base_r + z * CH, CH)])

    def orow(r, carry):
        ones[r, pl.ds(0, 16)] = zero16 + 1.0
        return carry

    lax.fori_loop(0, CH, orow, 0)
    plsc.subcore_barrier()

    cbase = c * NCTOT + s * NCHUNK
    cb = (cb0, cb1, cb2, cb3)
    isem = (is0, is1, is2, is3)
    ssem = (ss0, ss1)

    ihd = [pltpu.make_async_copy(cidx_hbm.at[cbase + i], cb[i % 4],
                                 isem[i % 4])
           for i in range(NCHUNK)]
    shd = [pltpu.make_async_copy(ones, acc.at[cb[i % 4].at[1]], ssem[i % 2])
           for i in range(NCHUNK)]

    ihd[0].start()
    ihd[1].start()

    for i in range(NCHUNK):
        ihd[i].wait()
        if i + 2 < NCHUNK:
            ihd[i + 2].start()
        if i >= 1:
            shd[i - 1].wait()
        shd[i].start(add=True)

    shd[NCHUNK - 1].wait()
    plsc.subcore_barrier()
    pltpu.sync_copy(acc.at[pl.ds(base_r, RPT)],
                    out_hbm.at[pl.ds(c * HALF_PAD + base_r, RPT)])


_cnt = pl.kernel(
    _cnt_body,
    out_type=jax.ShapeDtypeStruct((NC * HALF_PAD, CW), jnp.float32),
    mesh=_MESH,
    compiler_params=_SC_PARAMS,
    scratch_types=(
        [pltpu.VMEM((2, CH), jnp.int32)] * 4
        + [pltpu.VMEM((CH, CW), jnp.float32)]
        + [pltpu.VMEM_SHARED((HALF_PAD, CW), jnp.float32)]
        + [pltpu.SemaphoreType.DMA] * 6
    ),
)


def kernel(x, edge_index, W_l0, W_r0, b0, W_l1, W_r1, b1, W_l2, W_r2, b2):
    ei = edge_index.astype(jnp.int32)
    src, dst = ei[0], ei[1]
    cidx = _cidx(src, dst)
    cnt = _cnt(cidx).reshape(NC, HALF_PAD, CW)
    p, q = _pre(x, W_l0, W_r0, b0)
    acc = _agg(p, cidx).reshape(NC, HALF_PAD, D)
    p, q = _fuse(acc, cnt, q, W_l1, W_r1, b1)
    acc = _agg(p, cidx).reshape(NC, HALF_PAD, D)
    p, q = _fuse(acc, cnt, q, W_l2, W_r2, b2)
    acc = _agg(p, cidx).reshape(NC, HALF_PAD, D)
    return _comb(acc, cnt, q)


# speedup vs baseline: 5.1693x; 1.3049x over previous
import functools

import jax
import jax.numpy as jnp
from jax import lax
from jax.experimental import pallas as pl
from jax.experimental.pallas import tpu as pltpu
from jax.experimental.pallas import tpu_sc as plsc

N = 10000
E = 160000
D = 256
NC = 2
NS = 16
HALF = N // NC
RPT = 320
HALF_PAD = NS * RPT
EPT = E // NS
CH = 80
NCHUNK = EPT // CH
NCTOT = E // CH
BN = 1000
BR = 1000
CW = 16

_SC_PARAMS = pltpu.CompilerParams(use_tc_tiling_on_sc=False)
_MESH = plsc.VectorSubcoreMesh(core_axis_name="c", subcore_axis_name="s")


def _pre_body(h_ref, wl_ref, wr_ref, b_ref, p_ref, q_ref):
    h = h_ref[...]
    p_ref[...] = jnp.dot(h, wl_ref[...], preferred_element_type=jnp.float32)
    q_ref[...] = (jnp.dot(h, wr_ref[...], preferred_element_type=jnp.float32)
                  + b_ref[...])


def _pre(h, wl, wr, b):
    return pl.pallas_call(
        _pre_body,
        grid=(N // BN,),
        in_specs=[
            pl.BlockSpec((BN, D), lambda i: (i, 0)),
            pl.BlockSpec((D, D), lambda i: (0, 0)),
            pl.BlockSpec((D, D), lambda i: (0, 0)),
            pl.BlockSpec((1, D), lambda i: (0, 0)),
        ],
        out_specs=[
            pl.BlockSpec((BN, D), lambda i: (i, 0)),
            pl.BlockSpec((BN, D), lambda i: (i, 0)),
        ],
        out_shape=[
            jax.ShapeDtypeStruct((N, D), jnp.float32),
            jax.ShapeDtypeStruct((N, D), jnp.float32),
        ],
    )(h, wl, wr, b.reshape(1, D))


def _fuse_body(acc_ref, cnt_ref, q_ref, wl_ref, wr_ref, b_ref, p_ref, q2_ref):
    a = acc_ref[0].astype(jnp.float32)
    cnt = cnt_ref[0][:, :1]
    inv = 1.0 / jnp.maximum(cnt, 1.0)
    h = jnp.maximum(a * inv + q_ref[...], 0.0)
    p_ref[...] = jnp.dot(h, wl_ref[...], preferred_element_type=jnp.float32)
    q2_ref[...] = (jnp.dot(h, wr_ref[...], preferred_element_type=jnp.float32)
                   + b_ref[...])


def _fuse(acc, cnt, q, wl, wr, b):
    nb = HALF // BN
    return pl.pallas_call(
        _fuse_body,
        grid=(NC, nb),
        in_specs=[
            pl.BlockSpec((1, BN, D), lambda c, i: (c, i, 0)),
            pl.BlockSpec((1, BN, CW), lambda c, i: (c, i, 0)),
            pl.BlockSpec((BN, D), lambda c, i: (c * nb + i, 0)),
            pl.BlockSpec((D, D), lambda c, i: (0, 0)),
            pl.BlockSpec((D, D), lambda c, i: (0, 0)),
            pl.BlockSpec((1, D), lambda c, i: (0, 0)),
        ],
        out_specs=[
            pl.BlockSpec((BN, D), lambda c, i: (c * nb + i, 0)),
            pl.BlockSpec((BN, D), lambda c, i: (c * nb + i, 0)),
        ],
        out_shape=[
            jax.ShapeDtypeStruct((N, D), jnp.float32),
            jax.ShapeDtypeStruct((N, D), jnp.float32),
        ],
    )(acc, cnt, q, wl, wr, b.reshape(1, D))


def _comb_body(acc_ref, cnt_ref, q_ref, o_ref):
    a = acc_ref[0].astype(jnp.float32)
    cnt = cnt_ref[0][:, :1]
    inv = 1.0 / jnp.maximum(cnt, 1.0)
    o_ref[...] = a * inv + q_ref[...]


def _comb(acc, cnt, q):
    return pl.pallas_call(
        _comb_body,
        grid=(NC, HALF // BR),
        in_specs=[
            pl.BlockSpec((1, BR, D), lambda c, i: (c, i, 0)),
            pl.BlockSpec((1, BR, CW), lambda c, i: (c, i, 0)),
            pl.BlockSpec((BR, D), lambda c, i: (c * (HALF // BR) + i, 0)),
        ],
        out_specs=pl.BlockSpec((BR, D), lambda c, i: (c * (HALF // BR) + i, 0)),
        out_shape=jax.ShapeDtypeStruct((N, D), jnp.float32),
    )(acc, cnt, q)


def _cidx_body(s_ref, d_ref, o_ref):
    c = pl.program_id(0)
    dv = d_ref[...]
    loc = dv - c * HALF
    ok = (loc >= 0) & (loc < HALF)
    o_ref[0, :, 0, :] = s_ref[...]
    o_ref[0, :, 1, :] = jnp.where(ok, loc, HALF)


def _cidx(src, dst):
    out = pl.pallas_call(
        _cidx_body,
        grid=(NC,),
        in_specs=[
            pl.BlockSpec((NCTOT, CH), lambda c: (0, 0)),
            pl.BlockSpec((NCTOT, CH), lambda c: (0, 0)),
        ],
        out_specs=pl.BlockSpec((1, NCTOT, 2, CH), lambda c: (c, 0, 0, 0)),
        out_shape=jax.ShapeDtypeStruct((NC, NCTOT, 2, CH), jnp.int32),
    )(src.reshape(NCTOT, CH), dst.reshape(NCTOT, CH))
    return out.reshape(NC * NCTOT, 2, CH)


def _agg_body(p_hbm, cidx_hbm, out_hbm,
              cb0, cb1, cb2, cb3, rows0, rows1, acc,
              gs0, gs1, ss0, ss1, is0, is1, is2, is3):
    c = lax.axis_index("c")
    s = lax.axis_index("s")
    zero32 = jnp.zeros((32,), jnp.bfloat16)

    def zrow(r, carry):
        for j in range(D // 32):
            rows0[r, pl.ds(j * 32, 32)] = zero32
        return carry

    lax.fori_loop(0, CH, zrow, 0)
    base_r = s * RPT
    for z in range(RPT // CH):
        pltpu.sync_copy(rows0, acc.at[pl.ds(base_r + z * CH, CH)])
    plsc.subcore_barrier()

    cbase = c * NCTOT + s * NCHUNK
    cb = (cb0, cb1, cb2, cb3)
    isem = (is0, is1, is2, is3)
    rows = (rows0, rows1)
    gsem = (gs0, gs1)
    ssem = (ss0, ss1)

    ihd = [pltpu.make_async_copy(cidx_hbm.at[cbase + i], cb[i % 4],
                                 isem[i % 4])
           for i in range(NCHUNK)]
    ghd = [pltpu.make_async_copy(p_hbm.at[cb[i % 4].at[0]], rows[i % 2],
                                 gsem[i % 2])
           for i in range(NCHUNK)]
    shd = [pltpu.make_async_copy(rows[i % 2], acc.at[cb[i % 4].at[1]],
                                 ssem[i % 2])
           for i in range(NCHUNK)]

    ihd[0].start()
    ihd[1].start()
    ihd[0].wait()
    ghd[0].start()

    for i in range(NCHUNK):
        ghd[i].wait()
        if i + 2 < NCHUNK:
            ihd[i + 2].start()
        if i >= 1:
            shd[i - 1].wait()
        if i + 1 < NCHUNK:
            ihd[i + 1].wait()
            ghd[i + 1].start()
        shd[i].start(add=True)

    shd[NCHUNK - 1].wait()
    plsc.subcore_barrier()
    pltpu.sync_copy(acc.at[pl.ds(base_r, RPT)],
                    out_hbm.at[pl.ds(c * HALF_PAD + base_r, RPT)])


_agg = pl.kernel(
    _agg_body,
    out_type=jax.ShapeDtypeStruct((NC * HALF_PAD, D), jnp.bfloat16),
    mesh=_MESH,
    compiler_params=_SC_PARAMS,
    scratch_types=(
        [pltpu.VMEM((2, CH), jnp.int32)] * 4
        + [pltpu.VMEM((CH, D), jnp.bfloat16)] * 2
        + [pltpu.VMEM_SHARED((HALF_PAD, D), jnp.bfloat16)]
        + [pltpu.SemaphoreType.DMA] * 8
    ),
)


def _cnt_body(cidx_hbm, out_hbm, cb0, cb1, cb2, cb3, ones, acc,
              ss0, ss1, is0, is1, is2, is3):
    c = lax.axis_index("c")
    s = lax.axis_index("s")
    zero16 = jnp.zeros((16,), jnp.float32)

    def zrow(r, carry):
        ones[r, pl.ds(0, 16)] = zero16
        return carry

    lax.fori_loop(0, CH, zrow, 0)
    base_r = s * RPT
    for z in range(RPT // CH):
        pltpu.sync_copy(ones, acc.at[pl.ds(base_r + z * CH, CH)])

    def orow(r, carry):
        ones[r, pl.ds(0, 16)] = zero16 + 1.0
        return carry

    lax.fori_loop(0, CH, orow, 0)
    plsc.subcore_barrier()

    cbase = c * NCTOT + s * NCHUNK
    cb = (cb0, cb1, cb2, cb3)
    isem = (is0, is1, is2, is3)
    ssem = (ss0, ss1)

    ihd = [pltpu.make_async_copy(cidx_hbm.at[cbase + i], cb[i % 4],
                                 isem[i % 4])
           for i in range(NCHUNK)]
    shd = [pltpu.make_async_copy(ones, acc.at[cb[i % 4].at[1]], ssem[i % 2])
           for i in range(NCHUNK)]

    ihd[0].start()
    ihd[1].start()

    for i in range(NCHUNK):
        ihd[i].wait()
        if i + 2 < NCHUNK:
            ihd[i + 2].start()
        if i >= 1:
            shd[i - 1].wait()
        shd[i].start(add=True)

    shd[NCHUNK - 1].wait()
    plsc.subcore_barrier()
    pltpu.sync_copy(acc.at[pl.ds(base_r, RPT)],
                    out_hbm.at[pl.ds(c * HALF_PAD + base_r, RPT)])


_cnt = pl.kernel(
    _cnt_body,
    out_type=jax.ShapeDtypeStruct((NC * HALF_PAD, CW), jnp.float32),
    mesh=_MESH,
    compiler_params=_SC_PARAMS,
    scratch_types=(
        [pltpu.VMEM((2, CH), jnp.int32)] * 4
        + [pltpu.VMEM((CH, CW), jnp.float32)]
        + [pltpu.VMEM_SHARED((HALF_PAD, CW), jnp.float32)]
        + [pltpu.SemaphoreType.DMA] * 6
    ),
)


def kernel(x, edge_index, W_l0, W_r0, b0, W_l1, W_r1, b1, W_l2, W_r2, b2):
    ei = edge_index.astype(jnp.int32)
    src, dst = ei[0], ei[1]
    cidx = _cidx(src, dst)
    cnt = _cnt(cidx).reshape(NC, HALF_PAD, CW)
    p, q = _pre(x, W_l0, W_r0, b0)
    acc = _agg(p.astype(jnp.bfloat16), cidx).reshape(NC, HALF_PAD, D)
    p, q = _fuse(acc, cnt, q, W_l1, W_r1, b1)
    acc = _agg(p.astype(jnp.bfloat16), cidx).reshape(NC, HALF_PAD, D)
    p, q = _fuse(acc, cnt, q, W_l2, W_r2, b2)
    acc = _agg(p.astype(jnp.bfloat16), cidx).reshape(NC, HALF_PAD, D)
    return _comb(acc, cnt, q)
